# Initial kernel scaffold; baseline (speedup 1.0000x reference)
import jax
import jax.numpy as jnp
from jax.experimental import pallas as pl


def kernel(inputs, targets, features):
    raise NotImplementedError("write your pallas kernel here")


# speedup vs baseline: 3.6224x; 3.6224x over previous
import jax
import jax.numpy as jnp
from jax.experimental import pallas as pl
from jax.experimental.pallas import tpu as pltpu

B = 1024
F = 64
N = 100000
TOTAL = 2 * N
T = 2048
GRID = (TOTAL + T - 1) // T
S_TILE = N // T
LAST = GRID - 1
INV_TEMP = 1.0 / 0.05


def _tc_body(xT_ref, f_ref, tgt_ref, cem_ref, ceh_ref,
             off_ref, sm_ref, sh_ref, tm_ref, th_ref):
    i = pl.program_id(0)

    @pl.when(i == 0)
    def _init():
        xT0 = xT_ref[...]
        off_ref[...] = jnp.sqrt(jnp.sum(xT0 * xT0, axis=0, keepdims=True)) - 70.0
        zero = jnp.zeros((1, B), jnp.float32)
        sm_ref[...] = zero
        sh_ref[...] = zero
        tm_ref[...] = zero
        th_ref[...] = zero

    off = off_ref[...]
    tgt = tgt_ref[0:1, :]
    lt = jax.lax.dot_general(
        f_ref[...], xT_ref[...],
        (((1,), (0,)), ((), ())),
        preferred_element_type=jnp.float32)
    row = jax.lax.broadcasted_iota(jnp.int32, (T, 1), 0) + i * T
    p = jnp.exp(lt - off)

    @pl.when(i < S_TILE)
    def _mean_tile():
        sm_ref[...] += jnp.sum(p, axis=0, keepdims=True)
        tm_ref[...] += jnp.sum(jnp.where(row == tgt, lt, 0.0),
                               axis=0, keepdims=True)

    @pl.when(i == S_TILE)
    def _straddle_tile():
        mmask = row < N
        sm_ref[...] += jnp.sum(jnp.where(mmask, p, 0.0), axis=0, keepdims=True)
        sh_ref[...] += jnp.sum(jnp.where(mmask, 0.0, p), axis=0, keepdims=True)
        tm_ref[...] += jnp.sum(jnp.where(row == tgt, lt, 0.0),
                               axis=0, keepdims=True)
        th_ref[...] += jnp.sum(jnp.where(row == tgt + N, lt, 0.0),
                               axis=0, keepdims=True)

    @pl.when(jnp.logical_and(i > S_TILE, i < LAST))
    def _hard_tile():
        sh_ref[...] += jnp.sum(p, axis=0, keepdims=True)
        th_ref[...] += jnp.sum(jnp.where(row == tgt + N, lt, 0.0),
                               axis=0, keepdims=True)

    @pl.when(i == LAST)
    def _last_tile():
        vmask = row < TOTAL
        sh_ref[...] += jnp.sum(jnp.where(vmask, p, 0.0), axis=0, keepdims=True)
        th_ref[...] += jnp.sum(jnp.where(row == tgt + N, lt, 0.0),
                               axis=0, keepdims=True)
        logzm = off + jnp.log(sm_ref[...])
        logzh = off + jnp.log(sh_ref[...])
        cem = jnp.mean(logzm - tm_ref[...])
        ceh = jnp.mean(logzh - th_ref[...])
        cem_ref[...] = jnp.full((8, 128), cem, jnp.float32)
        ceh_ref[...] = jnp.full((8, 128), ceh, jnp.float32)


def _run_tc(xT, features, tgtb):
    return pl.pallas_call(
        _tc_body,
        grid=(GRID,),
        in_specs=[
            pl.BlockSpec((F, B), lambda i: (0, 0)),
            pl.BlockSpec((T, F), lambda i: (i, 0)),
            pl.BlockSpec((8, B), lambda i: (0, 0)),
        ],
        out_specs=[
            pl.BlockSpec((8, 128), lambda i: (0, 0)),
            pl.BlockSpec((8, 128), lambda i: (0, 0)),
        ],
        out_shape=[
            jax.ShapeDtypeStruct((8, 128), jnp.float32),
            jax.ShapeDtypeStruct((8, 128), jnp.float32),
        ],
        scratch_shapes=[
            pltpu.VMEM((1, B), jnp.float32),
            pltpu.VMEM((1, B), jnp.float32),
            pltpu.VMEM((1, B), jnp.float32),
            pltpu.VMEM((1, B), jnp.float32),
            pltpu.VMEM((1, B), jnp.float32),
        ],
        compiler_params=pltpu.CompilerParams(
            dimension_semantics=("arbitrary",)),
    )(xT, features, tgtb)


def kernel(inputs, targets, features):
    xT = (inputs * INV_TEMP).T
    tgtb = jnp.broadcast_to(targets.astype(jnp.int32)[None, :], (8, B))
    cem, ceh = _run_tc(xT, features, tgtb)
    ce_mean = cem[0, 0]
    ce_hard = ceh[0, 0]
    return 0.5 * (ce_hard + jnp.maximum(ce_mean - 0.2, 0.0))
